# baseline (speedup 1.0000x reference)
import jax
import jax.numpy as jnp
from jax import lax
from jax.experimental import pallas as pl
from jax.experimental.pallas import tpu as pltpu
from jax.experimental.pallas import tpu_sc as plsc

N = 10000
E = 320000
D = 128
HALF = D // 2
EPS = 1e-5

NC = 2
NS = 16
CHUNK = 128
NCHUNK = 157
EPT = NCHUNK * CHUNK
E_PAD = EPT * NS
NPAD = 10240
STRIPE = NPAD // NS
ZR = 64

_MESH = plsc.VectorSubcoreMesh(core_axis_name="c", subcore_axis_name="s")


DEG_SPLIT = NCHUNK // 2


def _make_sc_agg(compute_deg, NBUF, LOOK):
    SLACK = NBUF - LOOK
    out_type = [jax.ShapeDtypeStruct((NPAD, D), jnp.float32)]
    scratch = [
        pltpu.VMEM((NCHUNK, CHUNK), jnp.int32),
        pltpu.VMEM((NCHUNK, CHUNK), jnp.int32),
        pltpu.VMEM((NBUF, CHUNK, HALF), jnp.float32),
        pltpu.VMEM_SHARED((NPAD, HALF), jnp.float32),
    ] + [pltpu.SemaphoreType.DMA] * (2 * NBUF)
    if compute_deg:
        out_type.append(jax.ShapeDtypeStruct((NC * NPAD, 16), jnp.float32))
        scratch += [
            pltpu.VMEM((CHUNK, 16), jnp.float32),
            pltpu.VMEM_SHARED((NPAD, 16), jnp.float32),
            pltpu.SemaphoreType.DMA,
            pltpu.SemaphoreType.DMA,
        ]

    def body(h2n, srcpe, srcpo, dstp, zrows, zdeg, ones_h, *refs):
        if compute_deg:
            (agg_out, deg_out, src_v, dst_v, bufs, acc,
             *sems2) = refs
            sems = sems2[:2 * NBUF]
            ones_v, dacc, dsem0, dsem1 = sems2[2 * NBUF:]
            dsem = (dsem0, dsem1)
        else:
            agg_out, src_v, dst_v, bufs, acc, *sems = refs
        gsem = sems[:NBUF]
        ssem = sems[NBUF:]
        cid = lax.axis_index("c")
        sid = lax.axis_index("s")

        def zero_body(z, carry):
            pltpu.sync_copy(zrows, acc.at[pl.ds(sid * STRIPE + z * ZR, ZR)])
            return carry

        lax.fori_loop(0, STRIPE // ZR, zero_body, 0)
        if compute_deg:
            pltpu.sync_copy(zdeg, dacc.at[pl.ds(sid * STRIPE, STRIPE)])
            pltpu.sync_copy(ones_h, ones_v)
        plsc.subcore_barrier()

        @pl.when(cid == 0)
        def _():
            pltpu.sync_copy(srcpe.at[sid], src_v)

        @pl.when(cid == 1)
        def _():
            pltpu.sync_copy(srcpo.at[sid], src_v)

        pltpu.sync_copy(dstp.at[sid], dst_v)

        def start_gather(j, p):
            pltpu.async_copy(h2n.at[src_v.at[j]], bufs.at[p], gsem[p])

        def wait_gather(j, p):
            pltpu.make_async_copy(h2n.at[src_v.at[j]], bufs.at[p],
                                  gsem[p]).wait()

        def start_scatter(j, p):
            pltpu.async_copy(bufs.at[p], acc.at[dst_v.at[j]], ssem[p],
                             add=True)

        def wait_scatter(j, p):
            pltpu.make_async_copy(bufs.at[p], acc.at[dst_v.at[j]],
                                  ssem[p]).wait()

        def count_deg(k, par, first=False):
            if compute_deg:
                if not first:
                    pltpu.make_async_copy(ones_v, dacc.at[dst_v.at[k - 2]],
                                          dsem[par]).wait()
                pltpu.async_copy(ones_v, dacc.at[dst_v.at[k]], dsem[par],
                                 add=True)

        def step(k, p, q, par, prefetch):
            wait_gather(k, p)
            start_scatter(k, p)
            count_deg(k, par)
            if prefetch:
                wait_scatter(k - SLACK, q)
                start_gather(k + LOOK, q)

        for m in range(LOOK):
            start_gather(m, m)
        for k in range(SLACK):
            wait_gather(k, k)
            start_scatter(k, k)
            count_deg(k, k % 2, first=(k < 2))
            start_gather(k + LOOK, (k + LOOK) % NBUF)

        blocks = (NCHUNK - LOOK - SLACK) // NBUF
        rem = (NCHUNK - LOOK - SLACK) % NBUF

        def loop_body(ii, carry):
            for t in range(NBUF):
                k = SLACK + ii * NBUF + t
                step(k, (SLACK + t) % NBUF, t, (SLACK + t) % 2, True)
            return carry

        lax.fori_loop(0, blocks, loop_body, 0)
        for t in range(rem):
            k = SLACK + blocks * NBUF + t
            step(k, k % NBUF, (k - SLACK) % NBUF, k % 2, True)
        for e in range(LOOK):
            k = NCHUNK - LOOK + e
            wait_gather(k, k % NBUF)
            start_scatter(k, k % NBUF)
            count_deg(k, k % 2)
        for e in range(NBUF):
            k = NCHUNK - NBUF + e
            wait_scatter(k, k % NBUF)
        if compute_deg:
            for k in (NCHUNK - 2, NCHUNK - 1):
                pltpu.make_async_copy(ones_v, dacc.at[dst_v.at[k]],
                                      dsem[k % 2]).wait()

        plsc.subcore_barrier()

        @pl.when(cid == 0)
        def _():
            pltpu.sync_copy(acc.at[pl.ds(sid * STRIPE, STRIPE)],
                            agg_out.at[pl.ds(sid * STRIPE, STRIPE), 0:HALF])

        @pl.when(cid == 1)
        def _():
            pltpu.sync_copy(acc.at[pl.ds(sid * STRIPE, STRIPE)],
                            agg_out.at[pl.ds(sid * STRIPE, STRIPE), HALF:D])

        if compute_deg:
            row0 = cid * NPAD + sid * STRIPE
            pltpu.sync_copy(dacc.at[pl.ds(sid * STRIPE, STRIPE)],
                            deg_out.at[pl.ds(row0, STRIPE)])

    return pl.kernel(
        body,
        out_type=tuple(out_type) if compute_deg else out_type[0],
        mesh=_MESH,
        scratch_types=scratch,
        compiler_params=pltpu.CompilerParams(use_tc_tiling_on_sc=False),
    )


_sc_agg_deg = _make_sc_agg(True, 4, 2)
_sc_agg = _make_sc_agg(False, 6, 3)


def _dot_t(a, w):
    return lax.dot_general(a, w, (((1,), (1,)), ((), ())),
                           preferred_element_type=jnp.float32)


def _vn_mlp_math(v, W1, b1, g1, t1, W2, b2, g2, t2):
    h1 = jnp.maximum(_dot_t(v, W1) + b1, 0.0)
    mu = jnp.mean(h1, axis=1, keepdims=True)
    va = jnp.mean(h1 * h1, axis=1, keepdims=True) - mu * mu
    h1 = (h1 - mu) * lax.rsqrt(va + EPS) * g1 + t1
    h2 = jnp.maximum(_dot_t(h1, W2) + b2, 0.0)
    mu = jnp.mean(h2, axis=1, keepdims=True)
    va = jnp.mean(h2 * h2, axis=1, keepdims=True) - mu * mu
    return (h2 - mu) * lax.rsqrt(va + EPS) * g2 + t2


def _deg_cols(degp):
    deg = (degp[0:N, 0:1] + degp[NPAD:NPAD + N, 0:1]) * 0.5
    inv = 1.0 / jnp.maximum(deg, 1.0)
    return deg, inv, deg * inv


def _lin_of(h, aggp, inv, cvec, avn, Wl, bl, Wr):
    aggm = aggp[0:N, :] * inv + cvec * avn
    return _dot_t(aggm, Wl) + _dot_t(h, Wr) + bl + _dot_t(avn, Wr)


def _bn(lin, g, b, relu):
    mu = jnp.mean(lin, axis=0, keepdims=True)
    va = jnp.mean(lin * lin, axis=0, keepdims=True) - mu * mu
    out = (lin - mu) * lax.rsqrt(va + EPS) * g + b
    if relu:
        out = jnp.maximum(out, 0.0)
    return out


def _tc_fwd_body(h, aggp, degp, vn, Wl, bl, Wr, g, b,
                 W1, b1, g1, t1, W2, b2, g2, t2,
                 h_out, vn_out):
    _, inv, cvec = _deg_cols(degp)
    avn = vn[...]
    hh = h[...]
    lin = _lin_of(hh, aggp, inv, cvec, avn, Wl[...], bl[...], Wr[...])
    h_out[...] = _bn(lin, g[...], b[...], True)
    pooled = jnp.sum(hh, axis=0, keepdims=True)
    vn_out[...] = _vn_mlp_math(pooled + avn, W1[...], b1[...], g1[...], t1[...],
                               W2[...], b2[...], g2[...], t2[...])


def _tc_last_body(h, aggp, degp, vn, Wl, bl, Wr, g, b, h_out):
    _, inv, cvec = _deg_cols(degp)
    avn = vn[...]
    lin = _lin_of(h[...], aggp, inv, cvec, avn, Wl[...], bl[...], Wr[...])
    h_out[...] = _bn(lin, g[...], b[...], False)


_tc_fwd = pl.pallas_call(
    _tc_fwd_body,
    out_shape=(jax.ShapeDtypeStruct((N, D), jnp.float32),
               jax.ShapeDtypeStruct((1, D), jnp.float32)),
)

_tc_last = pl.pallas_call(
    _tc_last_body,
    out_shape=jax.ShapeDtypeStruct((N, D), jnp.float32),
)


def kernel(x, adj_t, params):
    src = adj_t[0].astype(jnp.int32)
    dst = adj_t[1].astype(jnp.int32)
    pad = E_PAD - E
    pad_ar = jnp.arange(pad, dtype=jnp.int32)
    src_p = jnp.concatenate([src, pad_ar % N])
    srcpe = (src_p * 2).reshape(NS, NCHUNK, CHUNK)
    srcpo = (src_p * 2 + 1).reshape(NS, NCHUNK, CHUNK)
    dstp = jnp.concatenate([dst, N + pad_ar % (NPAD - N)]).reshape(NS, NCHUNK, CHUNK)
    zrows = jnp.zeros((ZR, HALF), jnp.float32)
    zdeg = jnp.zeros((STRIPE, 16), jnp.float32)
    ones_h = jnp.ones((CHUNK, 16), jnp.float32)

    convs = params["convs"]
    bns = params["bns"]
    mlps = params["vn_mlps"]
    vn0 = params["vn"]

    def row(v):
        return v.reshape(1, -1)

    def mlp_args(m):
        return (m["W1"], row(m["b1"]), row(m["g1"]), row(m["t1"]),
                m["W2"], row(m["b2"]), row(m["g2"]), row(m["t2"]))

    def view2n(h):
        return h.reshape(2 * N, HALF)

    aggp, degp = _sc_agg_deg(view2n(x), srcpe, srcpo, dstp, zrows, zdeg, ones_h)
    h1, vn1 = _tc_fwd(
        x, aggp, degp, vn0,
        convs[0]["Wl"], row(convs[0]["bl"]), convs[0]["Wr"],
        row(bns[0]["g"]), row(bns[0]["b"]), *mlp_args(mlps[0]))

    aggp2 = _sc_agg(view2n(h1), srcpe, srcpo, dstp, zrows, zdeg, ones_h)
    h2, vn2 = _tc_fwd(
        h1, aggp2, degp, vn1,
        convs[1]["Wl"], row(convs[1]["bl"]), convs[1]["Wr"],
        row(bns[1]["g"]), row(bns[1]["b"]), *mlp_args(mlps[1]))

    aggp3 = _sc_agg(view2n(h2), srcpe, srcpo, dstp, zrows, zdeg, ones_h)
    h3 = _tc_last(
        h2, aggp3, degp, vn2,
        convs[2]["Wl"], row(convs[2]["bl"]), convs[2]["Wr"],
        row(bns[2]["g"]), row(bns[2]["b"]))
    return h3


# speedup vs baseline: 1.0072x; 1.0072x over previous
import jax
import jax.numpy as jnp
from jax import lax
from jax.experimental import pallas as pl
from jax.experimental.pallas import tpu as pltpu
from jax.experimental.pallas import tpu_sc as plsc

N = 10000
E = 320000
D = 128
HALF = D // 2
EPS = 1e-5

NC = 2
NS = 16
CHUNK = 128
NCHUNK = 157
EPT = NCHUNK * CHUNK
E_PAD = EPT * NS
NPAD = 10240
STRIPE = NPAD // NS
ZR = 64

_MESH = plsc.VectorSubcoreMesh(core_axis_name="c", subcore_axis_name="s")


DEG_SPLIT = NCHUNK // 2


def _make_sc_agg(compute_deg, NBUF, LOOK):
    SLACK = NBUF - LOOK
    out_type = [jax.ShapeDtypeStruct((NPAD, D), jnp.float32)]
    scratch = [
        pltpu.VMEM((NCHUNK, CHUNK), jnp.int32),
        pltpu.VMEM((NCHUNK, CHUNK), jnp.int32),
        pltpu.VMEM((NBUF, CHUNK, HALF), jnp.float32),
        pltpu.VMEM_SHARED((NPAD, HALF), jnp.float32),
    ] + [pltpu.SemaphoreType.DMA] * (2 * NBUF)
    if compute_deg:
        out_type.append(jax.ShapeDtypeStruct((NC * NPAD, 16), jnp.float32))
        scratch += [
            pltpu.VMEM((CHUNK, 16), jnp.float32),
            pltpu.VMEM_SHARED((NPAD, 16), jnp.float32),
            pltpu.SemaphoreType.DMA,
            pltpu.SemaphoreType.DMA,
        ]

    def body(h2n, srcpe, srcpo, dstp, zrows, zdeg, ones_h, *refs):
        if compute_deg:
            (agg_out, deg_out, src_v, dst_v, bufs, acc,
             *sems2) = refs
            sems = sems2[:2 * NBUF]
            ones_v, dacc, dsem0, dsem1 = sems2[2 * NBUF:]
            dsem = (dsem0, dsem1)
        else:
            agg_out, src_v, dst_v, bufs, acc, *sems = refs
        gsem = sems[:NBUF]
        ssem = sems[NBUF:]
        cid = lax.axis_index("c")
        sid = lax.axis_index("s")

        def zero_body(z, carry):
            pltpu.sync_copy(zrows, acc.at[pl.ds(sid * STRIPE + z * ZR, ZR)])
            return carry

        lax.fori_loop(0, STRIPE // ZR, zero_body, 0)
        if compute_deg:
            pltpu.sync_copy(zdeg, dacc.at[pl.ds(sid * STRIPE, STRIPE)])
            pltpu.sync_copy(ones_h, ones_v)
        plsc.subcore_barrier()

        @pl.when(cid == 0)
        def _():
            pltpu.sync_copy(srcpe.at[sid], src_v)

        @pl.when(cid == 1)
        def _():
            pltpu.sync_copy(srcpo.at[sid], src_v)

        pltpu.sync_copy(dstp.at[sid], dst_v)

        def start_gather(j, p):
            pltpu.async_copy(h2n.at[src_v.at[j]], bufs.at[p], gsem[p])

        def wait_gather(j, p):
            pltpu.make_async_copy(h2n.at[src_v.at[j]], bufs.at[p],
                                  gsem[p]).wait()

        def start_scatter(j, p):
            pltpu.async_copy(bufs.at[p], acc.at[dst_v.at[j]], ssem[p],
                             add=True)

        def wait_scatter(j, p):
            pltpu.make_async_copy(bufs.at[p], acc.at[dst_v.at[j]],
                                  ssem[p]).wait()

        def count_deg(k, par, first=False):
            if compute_deg:
                @pl.when(cid == par)
                def _():
                    if not first:
                        pltpu.make_async_copy(ones_v,
                                              dacc.at[dst_v.at[k - 2]],
                                              dsem[0]).wait()
                    pltpu.async_copy(ones_v, dacc.at[dst_v.at[k]], dsem[0],
                                     add=True)

        def step(k, p, q, par, prefetch):
            wait_gather(k, p)
            start_scatter(k, p)
            count_deg(k, par)
            if prefetch:
                wait_scatter(k - SLACK, q)
                start_gather(k + LOOK, q)

        for m in range(LOOK):
            start_gather(m, m)
        for k in range(SLACK):
            wait_gather(k, k)
            start_scatter(k, k)
            count_deg(k, k % 2, first=(k < 2))
            start_gather(k + LOOK, (k + LOOK) % NBUF)

        blocks = (NCHUNK - LOOK - SLACK) // NBUF
        rem = (NCHUNK - LOOK - SLACK) % NBUF

        def loop_body(ii, carry):
            for t in range(NBUF):
                k = SLACK + ii * NBUF + t
                step(k, (SLACK + t) % NBUF, t, (SLACK + t) % 2, True)
            return carry

        lax.fori_loop(0, blocks, loop_body, 0)
        for t in range(rem):
            k = SLACK + blocks * NBUF + t
            step(k, k % NBUF, (k - SLACK) % NBUF, k % 2, True)
        for e in range(LOOK):
            k = NCHUNK - LOOK + e
            wait_gather(k, k % NBUF)
            start_scatter(k, k % NBUF)
            count_deg(k, k % 2)
        for e in range(NBUF):
            k = NCHUNK - NBUF + e
            wait_scatter(k, k % NBUF)
        if compute_deg:
            for k in (NCHUNK - 2, NCHUNK - 1):
                @pl.when(cid == k % 2)
                def _(k=k):
                    pltpu.make_async_copy(ones_v, dacc.at[dst_v.at[k]],
                                          dsem[0]).wait()

        plsc.subcore_barrier()

        @pl.when(cid == 0)
        def _():
            pltpu.sync_copy(acc.at[pl.ds(sid * STRIPE, STRIPE)],
                            agg_out.at[pl.ds(sid * STRIPE, STRIPE), 0:HALF])

        @pl.when(cid == 1)
        def _():
            pltpu.sync_copy(acc.at[pl.ds(sid * STRIPE, STRIPE)],
                            agg_out.at[pl.ds(sid * STRIPE, STRIPE), HALF:D])

        if compute_deg:
            row0 = cid * NPAD + sid * STRIPE
            pltpu.sync_copy(dacc.at[pl.ds(sid * STRIPE, STRIPE)],
                            deg_out.at[pl.ds(row0, STRIPE)])

    return pl.kernel(
        body,
        out_type=tuple(out_type) if compute_deg else out_type[0],
        mesh=_MESH,
        scratch_types=scratch,
        compiler_params=pltpu.CompilerParams(use_tc_tiling_on_sc=False),
    )


_sc_agg_deg = _make_sc_agg(True, 4, 2)
_sc_agg = _make_sc_agg(False, 6, 3)


def _dot_t(a, w):
    return lax.dot_general(a, w, (((1,), (1,)), ((), ())),
                           preferred_element_type=jnp.float32)


def _vn_mlp_math(v, W1, b1, g1, t1, W2, b2, g2, t2):
    h1 = jnp.maximum(_dot_t(v, W1) + b1, 0.0)
    mu = jnp.mean(h1, axis=1, keepdims=True)
    va = jnp.mean(h1 * h1, axis=1, keepdims=True) - mu * mu
    h1 = (h1 - mu) * lax.rsqrt(va + EPS) * g1 + t1
    h2 = jnp.maximum(_dot_t(h1, W2) + b2, 0.0)
    mu = jnp.mean(h2, axis=1, keepdims=True)
    va = jnp.mean(h2 * h2, axis=1, keepdims=True) - mu * mu
    return (h2 - mu) * lax.rsqrt(va + EPS) * g2 + t2


def _deg_cols(degp):
    deg = degp[0:N, 0:1] + degp[NPAD:NPAD + N, 0:1]
    inv = 1.0 / jnp.maximum(deg, 1.0)
    return deg, inv, deg * inv


def _lin_of(h, aggp, inv, cvec, avn, Wl, bl, Wr):
    aggm = aggp[0:N, :] * inv + cvec * avn
    return _dot_t(aggm, Wl) + _dot_t(h, Wr) + bl + _dot_t(avn, Wr)


def _bn(lin, g, b, relu):
    mu = jnp.mean(lin, axis=0, keepdims=True)
    va = jnp.mean(lin * lin, axis=0, keepdims=True) - mu * mu
    out = (lin - mu) * lax.rsqrt(va + EPS) * g + b
    if relu:
        out = jnp.maximum(out, 0.0)
    return out


def _tc_fwd_body(h, aggp, degp, vn, Wl, bl, Wr, g, b,
                 W1, b1, g1, t1, W2, b2, g2, t2,
                 h_out, vn_out):
    _, inv, cvec = _deg_cols(degp)
    avn = vn[...]
    hh = h[...]
    lin = _lin_of(hh, aggp, inv, cvec, avn, Wl[...], bl[...], Wr[...])
    h_out[...] = _bn(lin, g[...], b[...], True)
    pooled = jnp.sum(hh, axis=0, keepdims=True)
    vn_out[...] = _vn_mlp_math(pooled + avn, W1[...], b1[...], g1[...], t1[...],
                               W2[...], b2[...], g2[...], t2[...])


def _tc_last_body(h, aggp, degp, vn, Wl, bl, Wr, g, b, h_out):
    _, inv, cvec = _deg_cols(degp)
    avn = vn[...]
    lin = _lin_of(h[...], aggp, inv, cvec, avn, Wl[...], bl[...], Wr[...])
    h_out[...] = _bn(lin, g[...], b[...], False)


_tc_fwd = pl.pallas_call(
    _tc_fwd_body,
    out_shape=(jax.ShapeDtypeStruct((N, D), jnp.float32),
               jax.ShapeDtypeStruct((1, D), jnp.float32)),
)

_tc_last = pl.pallas_call(
    _tc_last_body,
    out_shape=jax.ShapeDtypeStruct((N, D), jnp.float32),
)


def kernel(x, adj_t, params):
    src = adj_t[0].astype(jnp.int32)
    dst = adj_t[1].astype(jnp.int32)
    pad = E_PAD - E
    pad_ar = jnp.arange(pad, dtype=jnp.int32)
    src_p = jnp.concatenate([src, pad_ar % N])
    srcpe = (src_p * 2).reshape(NS, NCHUNK, CHUNK)
    srcpo = (src_p * 2 + 1).reshape(NS, NCHUNK, CHUNK)
    dstp = jnp.concatenate([dst, N + pad_ar % (NPAD - N)]).reshape(NS, NCHUNK, CHUNK)
    zrows = jnp.zeros((ZR, HALF), jnp.float32)
    zdeg = jnp.zeros((STRIPE, 16), jnp.float32)
    ones_h = jnp.ones((CHUNK, 16), jnp.float32)

    convs = params["convs"]
    bns = params["bns"]
    mlps = params["vn_mlps"]
    vn0 = params["vn"]

    def row(v):
        return v.reshape(1, -1)

    def mlp_args(m):
        return (m["W1"], row(m["b1"]), row(m["g1"]), row(m["t1"]),
                m["W2"], row(m["b2"]), row(m["g2"]), row(m["t2"]))

    def view2n(h):
        return h.reshape(2 * N, HALF)

    aggp, degp = _sc_agg_deg(view2n(x), srcpe, srcpo, dstp, zrows, zdeg, ones_h)
    h1, vn1 = _tc_fwd(
        x, aggp, degp, vn0,
        convs[0]["Wl"], row(convs[0]["bl"]), convs[0]["Wr"],
        row(bns[0]["g"]), row(bns[0]["b"]), *mlp_args(mlps[0]))

    aggp2 = _sc_agg(view2n(h1), srcpe, srcpo, dstp, zrows, zdeg, ones_h)
    h2, vn2 = _tc_fwd(
        h1, aggp2, degp, vn1,
        convs[1]["Wl"], row(convs[1]["bl"]), convs[1]["Wr"],
        row(bns[1]["g"]), row(bns[1]["b"]), *mlp_args(mlps[1]))

    aggp3 = _sc_agg(view2n(h2), srcpe, srcpo, dstp, zrows, zdeg, ones_h)
    h3 = _tc_last(
        h2, aggp3, degp, vn2,
        convs[2]["Wl"], row(convs[2]["bl"]), convs[2]["Wr"],
        row(bns[2]["g"]), row(bns[2]["b"]))
    return h3


# speedup vs baseline: 1.0157x; 1.0084x over previous
import jax
import jax.numpy as jnp
from jax import lax
from jax.experimental import pallas as pl
from jax.experimental.pallas import tpu as pltpu
from jax.experimental.pallas import tpu_sc as plsc

N = 10000
E = 320000
D = 128
HALF = D // 2
EPS = 1e-5

NC = 2
NS = 16
CHUNK = 128
NCHUNK = 157
EPT = NCHUNK * CHUNK
E_PAD = EPT * NS
NPAD = 10240
STRIPE = NPAD // NS
ZR = 64

_MESH = plsc.VectorSubcoreMesh(core_axis_name="c", subcore_axis_name="s")


DEG_SPLIT = NCHUNK // 2


def _make_sc_agg(compute_deg, NBUF, LOOK):
    SLACK = NBUF - LOOK
    out_type = [jax.ShapeDtypeStruct((NPAD, D), jnp.float32)]
    scratch = [
        pltpu.VMEM((NCHUNK, CHUNK), jnp.int32),
        pltpu.VMEM((NCHUNK, CHUNK), jnp.int32),
        pltpu.VMEM((NBUF, CHUNK, HALF), jnp.float32),
        pltpu.VMEM_SHARED((NPAD, HALF), jnp.float32),
    ] + [pltpu.SemaphoreType.DMA] * (2 * NBUF)
    if compute_deg:
        out_type.append(jax.ShapeDtypeStruct((NC * NPAD, 16), jnp.float32))
        scratch += [
            pltpu.VMEM((CHUNK, 16), jnp.float32),
            pltpu.VMEM_SHARED((NPAD, 16), jnp.float32),
        ]

    def body(h2n, srcpe, srcpo, dstp, zrows, zdeg, ones_h, *refs):
        if compute_deg:
            (agg_out, deg_out, src_v, dst_v, bufs, acc,
             *sems2) = refs
            sems = sems2[:2 * NBUF]
            ones_v, dacc = sems2[2 * NBUF:]
        else:
            agg_out, src_v, dst_v, bufs, acc, *sems = refs
        gsem = sems[:NBUF]
        ssem = sems[NBUF:]
        cid = lax.axis_index("c")
        sid = lax.axis_index("s")

        def zero_body(z, carry):
            pltpu.sync_copy(zrows, acc.at[pl.ds(sid * STRIPE + z * ZR, ZR)])
            return carry

        lax.fori_loop(0, STRIPE // ZR, zero_body, 0)
        if compute_deg:
            pltpu.sync_copy(zdeg, dacc.at[pl.ds(sid * STRIPE, STRIPE)])
            pltpu.sync_copy(ones_h, ones_v)
        plsc.subcore_barrier()

        @pl.when(cid == 0)
        def _():
            pltpu.sync_copy(srcpe.at[sid], src_v)

        @pl.when(cid == 1)
        def _():
            pltpu.sync_copy(srcpo.at[sid], src_v)

        pltpu.sync_copy(dstp.at[sid], dst_v)

        def start_gather(j, p):
            pltpu.async_copy(h2n.at[src_v.at[j]], bufs.at[p], gsem[p])

        def wait_gather(j, p):
            pltpu.make_async_copy(h2n.at[src_v.at[j]], bufs.at[p],
                                  gsem[p]).wait()

        def start_scatter(j, p):
            pltpu.async_copy(bufs.at[p], acc.at[dst_v.at[j]], ssem[p],
                             add=True)

        def wait_scatter(j, p):
            pltpu.make_async_copy(bufs.at[p], acc.at[dst_v.at[j]],
                                  ssem[p]).wait()

        def count_deg(k):
            if compute_deg:
                cond = jnp.where(cid == 0, k < DEG_SPLIT, k >= DEG_SPLIT)

                @pl.when(cond)
                def _():
                    pltpu.sync_copy(ones_v, dacc.at[dst_v.at[k]], add=True)

        def step(k, p, q, prefetch):
            wait_gather(k, p)
            start_scatter(k, p)
            count_deg(k)
            if prefetch:
                wait_scatter(k - SLACK, q)
                start_gather(k + LOOK, q)

        for m in range(LOOK):
            start_gather(m, m)
        for k in range(SLACK):
            wait_gather(k, k)
            start_scatter(k, k)
            count_deg(k)
            start_gather(k + LOOK, (k + LOOK) % NBUF)

        blocks = (NCHUNK - LOOK - SLACK) // NBUF
        rem = (NCHUNK - LOOK - SLACK) % NBUF

        def loop_body(ii, carry):
            for t in range(NBUF):
                k = SLACK + ii * NBUF + t
                step(k, (SLACK + t) % NBUF, t, True)
            return carry

        lax.fori_loop(0, blocks, loop_body, 0)
        for t in range(rem):
            k = SLACK + blocks * NBUF + t
            step(k, k % NBUF, (k - SLACK) % NBUF, True)
        for e in range(LOOK):
            k = NCHUNK - LOOK + e
            wait_gather(k, k % NBUF)
            start_scatter(k, k % NBUF)
            count_deg(k)
        for e in range(NBUF):
            k = NCHUNK - NBUF + e
            wait_scatter(k, k % NBUF)
        plsc.subcore_barrier()

        @pl.when(cid == 0)
        def _():
            pltpu.sync_copy(acc.at[pl.ds(sid * STRIPE, STRIPE)],
                            agg_out.at[pl.ds(sid * STRIPE, STRIPE), 0:HALF])

        @pl.when(cid == 1)
        def _():
            pltpu.sync_copy(acc.at[pl.ds(sid * STRIPE, STRIPE)],
                            agg_out.at[pl.ds(sid * STRIPE, STRIPE), HALF:D])

        if compute_deg:
            row0 = cid * NPAD + sid * STRIPE
            pltpu.sync_copy(dacc.at[pl.ds(sid * STRIPE, STRIPE)],
                            deg_out.at[pl.ds(row0, STRIPE)])

    return pl.kernel(
        body,
        out_type=tuple(out_type) if compute_deg else out_type[0],
        mesh=_MESH,
        scratch_types=scratch,
        compiler_params=pltpu.CompilerParams(use_tc_tiling_on_sc=False),
    )


_sc_agg_deg = _make_sc_agg(True, 4, 2)
_sc_agg = _make_sc_agg(False, 6, 3)


def _dot_t(a, w):
    return lax.dot_general(a, w, (((1,), (1,)), ((), ())),
                           preferred_element_type=jnp.float32)


def _vn_mlp_math(v, W1, b1, g1, t1, W2, b2, g2, t2):
    h1 = jnp.maximum(_dot_t(v, W1) + b1, 0.0)
    mu = jnp.mean(h1, axis=1, keepdims=True)
    va = jnp.mean(h1 * h1, axis=1, keepdims=True) - mu * mu
    h1 = (h1 - mu) * lax.rsqrt(va + EPS) * g1 + t1
    h2 = jnp.maximum(_dot_t(h1, W2) + b2, 0.0)
    mu = jnp.mean(h2, axis=1, keepdims=True)
    va = jnp.mean(h2 * h2, axis=1, keepdims=True) - mu * mu
    return (h2 - mu) * lax.rsqrt(va + EPS) * g2 + t2


def _deg_cols(degp):
    deg = degp[0:N, 0:1] + degp[NPAD:NPAD + N, 0:1]
    inv = 1.0 / jnp.maximum(deg, 1.0)
    return deg, inv, deg * inv


def _lin_of(h, aggp, inv, cvec, avn, Wl, bl, Wr):
    aggm = aggp[0:N, :] * inv + cvec * avn
    return _dot_t(aggm, Wl) + _dot_t(h, Wr) + bl + _dot_t(avn, Wr)


def _bn(lin, g, b, relu):
    mu = jnp.mean(lin, axis=0, keepdims=True)
    va = jnp.mean(lin * lin, axis=0, keepdims=True) - mu * mu
    out = (lin - mu) * lax.rsqrt(va + EPS) * g + b
    if relu:
        out = jnp.maximum(out, 0.0)
    return out


def _tc_fwd_body(h, aggp, degp, vn, Wl, bl, Wr, g, b,
                 W1, b1, g1, t1, W2, b2, g2, t2,
                 h_out, vn_out):
    _, inv, cvec = _deg_cols(degp)
    avn = vn[...]
    hh = h[...]
    lin = _lin_of(hh, aggp, inv, cvec, avn, Wl[...], bl[...], Wr[...])
    h_out[...] = _bn(lin, g[...], b[...], True)
    pooled = jnp.sum(hh, axis=0, keepdims=True)
    vn_out[...] = _vn_mlp_math(pooled + avn, W1[...], b1[...], g1[...], t1[...],
                               W2[...], b2[...], g2[...], t2[...])


def _tc_last_body(h, aggp, degp, vn, Wl, bl, Wr, g, b, h_out):
    _, inv, cvec = _deg_cols(degp)
    avn = vn[...]
    lin = _lin_of(h[...], aggp, inv, cvec, avn, Wl[...], bl[...], Wr[...])
    h_out[...] = _bn(lin, g[...], b[...], False)


_tc_fwd = pl.pallas_call(
    _tc_fwd_body,
    out_shape=(jax.ShapeDtypeStruct((N, D), jnp.float32),
               jax.ShapeDtypeStruct((1, D), jnp.float32)),
)

_tc_last = pl.pallas_call(
    _tc_last_body,
    out_shape=jax.ShapeDtypeStruct((N, D), jnp.float32),
)


def kernel(x, adj_t, params):
    src = adj_t[0].astype(jnp.int32)
    dst = adj_t[1].astype(jnp.int32)
    pad = E_PAD - E
    pad_ar = jnp.arange(pad, dtype=jnp.int32)
    src_p = jnp.concatenate([src, pad_ar % N])
    srcpe = (src_p * 2).reshape(NS, NCHUNK, CHUNK)
    srcpo = (src_p * 2 + 1).reshape(NS, NCHUNK, CHUNK)
    dstp = jnp.concatenate([dst, N + pad_ar % (NPAD - N)]).reshape(NS, NCHUNK, CHUNK)
    zrows = jnp.zeros((ZR, HALF), jnp.float32)
    zdeg = jnp.zeros((STRIPE, 16), jnp.float32)
    ones_h = jnp.ones((CHUNK, 16), jnp.float32)

    convs = params["convs"]
    bns = params["bns"]
    mlps = params["vn_mlps"]
    vn0 = params["vn"]

    def row(v):
        return v.reshape(1, -1)

    def mlp_args(m):
        return (m["W1"], row(m["b1"]), row(m["g1"]), row(m["t1"]),
                m["W2"], row(m["b2"]), row(m["g2"]), row(m["t2"]))

    def view2n(h):
        return h.reshape(2 * N, HALF)

    aggp, degp = _sc_agg_deg(view2n(x), srcpe, srcpo, dstp, zrows, zdeg, ones_h)
    h1, vn1 = _tc_fwd(
        x, aggp, degp, vn0,
        convs[0]["Wl"], row(convs[0]["bl"]), convs[0]["Wr"],
        row(bns[0]["g"]), row(bns[0]["b"]), *mlp_args(mlps[0]))

    aggp2 = _sc_agg(view2n(h1), srcpe, srcpo, dstp, zrows, zdeg, ones_h)
    h2, vn2 = _tc_fwd(
        h1, aggp2, degp, vn1,
        convs[1]["Wl"], row(convs[1]["bl"]), convs[1]["Wr"],
        row(bns[1]["g"]), row(bns[1]["b"]), *mlp_args(mlps[1]))

    aggp3 = _sc_agg(view2n(h2), srcpe, srcpo, dstp, zrows, zdeg, ones_h)
    h3 = _tc_last(
        h2, aggp3, degp, vn2,
        convs[2]["Wl"], row(convs[2]["bl"]), convs[2]["Wr"],
        row(bns[2]["g"]), row(bns[2]["b"]))
    return h3
